# Initial kernel scaffold; baseline (speedup 1.0000x reference)
import jax
import jax.numpy as jnp
from jax.experimental import pallas as pl


def kernel(x, adj, W, b):
    raise NotImplementedError("write your pallas kernel here")


# speedup vs baseline: 5.2806x; 5.2806x over previous
import functools

import jax
import jax.numpy as jnp
from jax import lax
from jax.experimental import pallas as pl
from jax.experimental.pallas import tpu as pltpu
from jax.experimental.pallas import tpu_sc as plsc

N = 10000
E = 320000
D = 128

NC = 2
NS = 16
CH = 80
N_PAD = 10240

E_PER_CORE = E // NC
E_PER_TILE = E_PER_CORE // NS
N_CHUNKS = E_PER_TILE // CH
ROWS_PER_TILE = N_PAD // NS


def _mm_body(x_ref, w_ref, o_ref):
    o_ref[...] = jnp.dot(x_ref[...], w_ref[...],
                         preferred_element_type=jnp.float32)


def _matmul(x_pad, W):
    blk = 1024
    return pl.pallas_call(
        _mm_body,
        grid=(N_PAD // blk,),
        in_specs=[
            pl.BlockSpec((blk, D), lambda i: (i, 0)),
            pl.BlockSpec((D, D), lambda i: (0, 0)),
        ],
        out_specs=pl.BlockSpec((blk, D), lambda i: (i, 0)),
        out_shape=jax.ShapeDtypeStruct((N_PAD, D), jnp.float32),
    )(x_pad, W)


def _seg_body(support_hbm, row_hbm, col_hbm, btile_hbm, out_hbm,
              row_v, col_v, rows_v, agg_sh, sem):
    cid = lax.axis_index("c")
    sid = lax.axis_index("s")

    r0 = sid * ROWS_PER_TILE
    @pl.when(cid == 0)
    def _():
        pltpu.sync_copy(support_hbm.at[pl.ds(r0, ROWS_PER_TILE)],
                        agg_sh.at[pl.ds(r0, ROWS_PER_TILE)])

    @pl.when(cid == 1)
    def _():
        pltpu.sync_copy(btile_hbm, agg_sh.at[pl.ds(r0, ROWS_PER_TILE)])

    plsc.subcore_barrier()

    base = cid * E_PER_CORE + sid * E_PER_TILE

    def body(i, _):
        eb = base + i * CH
        pltpu.sync_copy(row_hbm.at[pl.ds(eb, CH)], row_v)
        pltpu.sync_copy(col_hbm.at[pl.ds(eb, CH)], col_v)
        pltpu.async_copy(support_hbm.at[row_v], rows_v, sem).wait()
        pltpu.sync_copy(rows_v, agg_sh.at[col_v], add=True)
        return _

    lax.fori_loop(0, N_CHUNKS, body, None)

    plsc.subcore_barrier()

    pltpu.sync_copy(agg_sh.at[pl.ds(r0, ROWS_PER_TILE)],
                    out_hbm.at[cid, pl.ds(r0, ROWS_PER_TILE)])


def _segment_partials(support, row, col, btile):
    mesh = plsc.VectorSubcoreMesh(core_axis_name="c", subcore_axis_name="s")
    kern = functools.partial(
        pl.kernel,
        mesh=mesh,
        out_type=jax.ShapeDtypeStruct((NC, N_PAD, D), jnp.float32),
        scratch_types=[
            pltpu.VMEM((CH,), jnp.int32),
            pltpu.VMEM((CH,), jnp.int32),
            pltpu.VMEM((CH, D), jnp.float32),
            pltpu.VMEM_SHARED((N_PAD, D), jnp.float32),
            pltpu.SemaphoreType.DMA,
        ],
    )(_seg_body)
    return kern(support, row, col, btile)


def _comb_body(p_ref, o_ref):
    o_ref[...] = p_ref[0] + p_ref[1]


def _combine(partials):
    blk = 1024
    return pl.pallas_call(
        _comb_body,
        grid=(N_PAD // blk,),
        in_specs=[pl.BlockSpec((NC, blk, D), lambda i: (0, i, 0))],
        out_specs=pl.BlockSpec((blk, D), lambda i: (i, 0)),
        out_shape=jax.ShapeDtypeStruct((N_PAD, D), jnp.float32),
    )(partials)


def kernel(x, adj, W, b):
    row = adj[0].astype(jnp.int32)
    col = adj[1].astype(jnp.int32)
    x_pad = jnp.pad(x, ((0, N_PAD - N), (0, 0)))
    support = _matmul(x_pad, W)
    btile = jnp.broadcast_to(b[None, :], (ROWS_PER_TILE, D))
    partials = _segment_partials(support, row, col, btile)
    return _combine(partials)[:N]


# speedup vs baseline: 5.3943x; 1.0215x over previous
import functools

import jax
import jax.numpy as jnp
from jax import lax
from jax.experimental import pallas as pl
from jax.experimental.pallas import tpu as pltpu
from jax.experimental.pallas import tpu_sc as plsc

N = 10000
E = 320000
D = 128

NC = 2
NS = 16
CH = 80
N_PAD = 10240

E_PER_CORE = E // NC
E_PER_TILE = E_PER_CORE // NS
N_CHUNKS = E_PER_TILE // CH
ROWS_PER_TILE = N_PAD // NS


def _seg_body(x_hbm, row_hbm, col_hbm, out_hbm,
              row_v, col_v, rows_v, agg_sh, sem):
    cid = lax.axis_index("c")
    sid = lax.axis_index("s")

    r0 = sid * ROWS_PER_TILE
    pltpu.sync_copy(x_hbm.at[pl.ds(r0, ROWS_PER_TILE)],
                    agg_sh.at[pl.ds(r0, ROWS_PER_TILE)])

    plsc.subcore_barrier()

    base = cid * E_PER_CORE + sid * E_PER_TILE

    def body(i, _):
        eb = base + i * CH
        pltpu.sync_copy(row_hbm.at[pl.ds(eb, CH)], row_v)
        pltpu.sync_copy(col_hbm.at[pl.ds(eb, CH)], col_v)
        pltpu.async_copy(x_hbm.at[row_v], rows_v, sem).wait()
        pltpu.sync_copy(rows_v, agg_sh.at[col_v], add=True)
        return _

    lax.fori_loop(0, N_CHUNKS, body, None)

    plsc.subcore_barrier()

    pltpu.sync_copy(agg_sh.at[pl.ds(r0, ROWS_PER_TILE)],
                    out_hbm.at[cid, pl.ds(r0, ROWS_PER_TILE)])


def _segment_partials(x_pad, row, col):
    mesh = plsc.VectorSubcoreMesh(core_axis_name="c", subcore_axis_name="s")
    kern = functools.partial(
        pl.kernel,
        mesh=mesh,
        out_type=jax.ShapeDtypeStruct((NC, N_PAD, D), jnp.float32),
        scratch_types=[
            pltpu.VMEM((CH,), jnp.int32),
            pltpu.VMEM((CH,), jnp.int32),
            pltpu.VMEM((CH, D), jnp.float32),
            pltpu.VMEM_SHARED((N_PAD, D), jnp.float32),
            pltpu.SemaphoreType.DMA,
        ],
    )(_seg_body)
    return kern(x_pad, row, col)


def _fin_body(p_ref, x_ref, w_ref, b_ref, o_ref):
    acc = p_ref[0] + p_ref[1] - x_ref[...]
    o_ref[...] = jnp.dot(acc, w_ref[...],
                         preferred_element_type=jnp.float32) + b_ref[...]


def _finish(partials, x_pad, W, b):
    blk = 1024
    return pl.pallas_call(
        _fin_body,
        grid=(N_PAD // blk,),
        in_specs=[
            pl.BlockSpec((NC, blk, D), lambda i: (0, i, 0)),
            pl.BlockSpec((blk, D), lambda i: (i, 0)),
            pl.BlockSpec((D, D), lambda i: (0, 0)),
            pl.BlockSpec((1, D), lambda i: (0, 0)),
        ],
        out_specs=pl.BlockSpec((blk, D), lambda i: (i, 0)),
        out_shape=jax.ShapeDtypeStruct((N_PAD, D), jnp.float32),
    )(partials, x_pad, W, b)


def kernel(x, adj, W, b):
    row = adj[0].astype(jnp.int32)
    col = adj[1].astype(jnp.int32)
    x_pad = jnp.pad(x, ((0, N_PAD - N), (0, 0)))
    partials = _segment_partials(x_pad, row, col)
    return _finish(partials, x_pad, W, b[None, :])[:N]


# speedup vs baseline: 8.3588x; 1.5496x over previous
import functools

import jax
import jax.numpy as jnp
from jax import lax
from jax.experimental import pallas as pl
from jax.experimental.pallas import tpu as pltpu
from jax.experimental.pallas import tpu_sc as plsc

N = 10000
E = 320000
D = 128

NC = 2
NS = 16
CH = 80
N_PAD = 10240

E_PER_CORE = E // NC
E_PER_TILE = E_PER_CORE // NS
N_CHUNKS = E_PER_TILE // CH
ROWS_PER_TILE = N_PAD // NS


RING = 3


def _seg_body(x_hbm, row_hbm, col_hbm, out_hbm,
              r0v, r1v, r2v, c0v, c1v, c2v, b0, b1, b2, agg_sh,
              s0, s1, s2):
    cid = lax.axis_index("c")
    sid = lax.axis_index("s")

    r0 = sid * ROWS_PER_TILE
    pltpu.sync_copy(x_hbm.at[pl.ds(r0, ROWS_PER_TILE)],
                    agg_sh.at[pl.ds(r0, ROWS_PER_TILE)])

    plsc.subcore_barrier()

    base = cid * E_PER_CORE + sid * E_PER_TILE
    rows = (r0v, r1v, r2v)
    cols = (c0v, c1v, c2v)
    bufs = (b0, b1, b2)
    sems = (s0, s1, s2)

    gd = []
    for k in range(RING):
        eb = base + k * CH
        pltpu.sync_copy(row_hbm.at[pl.ds(eb, CH)], rows[k])
        pltpu.sync_copy(col_hbm.at[pl.ds(eb, CH)], cols[k])
        gd.append(pltpu.async_copy(x_hbm.at[rows[k]], bufs[k], sems[k]))

    def body(j, _):
        for t in range(RING):
            i = RING * j + t
            gd[t].wait()
            pltpu.sync_copy(bufs[t], agg_sh.at[cols[t]], add=True)
            nxt = i + RING

            @pl.when(nxt < N_CHUNKS)
            def _():
                eb = base + nxt * CH
                pltpu.sync_copy(row_hbm.at[pl.ds(eb, CH)], rows[t])
                pltpu.sync_copy(col_hbm.at[pl.ds(eb, CH)], cols[t])
                pltpu.async_copy(x_hbm.at[rows[t]], bufs[t], sems[t])
        return _

    lax.fori_loop(0, N_CHUNKS // RING, body, None)

    for t in range(N_CHUNKS % RING):
        gd[t].wait()
        pltpu.sync_copy(bufs[t], agg_sh.at[cols[t]], add=True)

    plsc.subcore_barrier()

    pltpu.sync_copy(agg_sh.at[pl.ds(r0, ROWS_PER_TILE)],
                    out_hbm.at[cid, pl.ds(r0, ROWS_PER_TILE)])


def _segment_partials(x_pad, row, col):
    mesh = plsc.VectorSubcoreMesh(core_axis_name="c", subcore_axis_name="s")
    kern = functools.partial(
        pl.kernel,
        mesh=mesh,
        out_type=jax.ShapeDtypeStruct((NC, N_PAD, D), jnp.float32),
        scratch_types=[
            pltpu.VMEM((CH,), jnp.int32),
            pltpu.VMEM((CH,), jnp.int32),
            pltpu.VMEM((CH,), jnp.int32),
            pltpu.VMEM((CH,), jnp.int32),
            pltpu.VMEM((CH,), jnp.int32),
            pltpu.VMEM((CH,), jnp.int32),
            pltpu.VMEM((CH, D), jnp.float32),
            pltpu.VMEM((CH, D), jnp.float32),
            pltpu.VMEM((CH, D), jnp.float32),
            pltpu.VMEM_SHARED((N_PAD, D), jnp.float32),
            pltpu.SemaphoreType.DMA,
            pltpu.SemaphoreType.DMA,
            pltpu.SemaphoreType.DMA,
        ],
    )(_seg_body)
    return kern(x_pad, row, col)


def _fin_body(p_ref, x_ref, w_ref, b_ref, o_ref):
    acc = p_ref[0] + p_ref[1] - x_ref[...]
    o_ref[...] = jnp.dot(acc, w_ref[...],
                         preferred_element_type=jnp.float32) + b_ref[...]


def _finish(partials, x_pad, W, b):
    blk = 1024
    return pl.pallas_call(
        _fin_body,
        grid=(N_PAD // blk,),
        in_specs=[
            pl.BlockSpec((NC, blk, D), lambda i: (0, i, 0)),
            pl.BlockSpec((blk, D), lambda i: (i, 0)),
            pl.BlockSpec((D, D), lambda i: (0, 0)),
            pl.BlockSpec((1, D), lambda i: (0, 0)),
        ],
        out_specs=pl.BlockSpec((blk, D), lambda i: (i, 0)),
        out_shape=jax.ShapeDtypeStruct((N_PAD, D), jnp.float32),
    )(partials, x_pad, W, b)


def kernel(x, adj, W, b):
    row = adj[0].astype(jnp.int32)
    col = adj[1].astype(jnp.int32)
    x_pad = jnp.pad(x, ((0, N_PAD - N), (0, 0)))
    partials = _segment_partials(x_pad, row, col)
    return _finish(partials, x_pad, W, b[None, :])[:N]


# speedup vs baseline: 13.6156x; 1.6289x over previous
import functools

import jax
import jax.numpy as jnp
from jax import lax
from jax.experimental import pallas as pl
from jax.experimental.pallas import tpu as pltpu
from jax.experimental.pallas import tpu_sc as plsc

N = 10000
E = 320000
D = 128

NC = 2
NS = 16
CH = 80
N_PAD = 10240

E_PER_CORE = E // NC
E_PER_TILE = E_PER_CORE // NS
N_CHUNKS = E_PER_TILE // CH
ROWS_PER_TILE = N_PAD // NS


RING = 4


def _seg_body(x_hbm, row_hbm, col_hbm, out_hbm,
              r0v, r1v, r2v, r3v, c0v, c1v, c2v, c3v,
              b0, b1, b2, b3, agg_sh,
              sg0, sg1, sg2, sg3, sr0, sr1, sr2, sr3,
              sc0, sc1, sc2, sc3):
    cid = lax.axis_index("c")
    sid = lax.axis_index("s")

    r0 = sid * ROWS_PER_TILE
    pltpu.sync_copy(x_hbm.at[pl.ds(r0, ROWS_PER_TILE)],
                    agg_sh.at[pl.ds(r0, ROWS_PER_TILE)])

    plsc.subcore_barrier()

    base = cid * E_PER_CORE + sid * E_PER_TILE
    rows = (r0v, r1v, r2v, r3v)
    cols = (c0v, c1v, c2v, c3v)
    bufs = (b0, b1, b2, b3)
    gsem = (sg0, sg1, sg2, sg3)
    rsem = (sr0, sr1, sr2, sr3)
    csem = (sc0, sc1, sc2, sc3)

    gd, cd = [], []
    for k in range(RING):
        eb = base + k * CH
        pltpu.sync_copy(row_hbm.at[pl.ds(eb, CH)], rows[k])
        gd.append(pltpu.async_copy(x_hbm.at[rows[k]], bufs[k], gsem[k]))
        cd.append(pltpu.async_copy(col_hbm.at[pl.ds(eb, CH)], cols[k],
                                   csem[k]))

    def body(j, _):
        for t in range(RING):
            i = RING * j + t
            nxt = i + RING
            eb = base + nxt * CH
            live = nxt < N_CHUNKS

            gd[t].wait()
            cd[t].wait()

            @pl.when(live)
            def _():
                pltpu.async_copy(row_hbm.at[pl.ds(eb, CH)], rows[t],
                                 rsem[t])

            pltpu.sync_copy(bufs[t], agg_sh.at[cols[t]], add=True)

            @pl.when(live)
            def _():
                pltpu.make_async_copy(row_hbm.at[pl.ds(eb, CH)], rows[t],
                                      rsem[t]).wait()
                pltpu.async_copy(x_hbm.at[rows[t]], bufs[t], gsem[t])
                pltpu.async_copy(col_hbm.at[pl.ds(eb, CH)], cols[t],
                                 csem[t])
        return _

    lax.fori_loop(0, N_CHUNKS // RING, body, None)

    for t in range(N_CHUNKS % RING):
        gd[t].wait()
        cd[t].wait()
        pltpu.sync_copy(bufs[t], agg_sh.at[cols[t]], add=True)

    plsc.subcore_barrier()

    pltpu.sync_copy(agg_sh.at[pl.ds(r0, ROWS_PER_TILE)],
                    out_hbm.at[cid, pl.ds(r0, ROWS_PER_TILE)])


def _segment_partials(x_pad, row, col):
    mesh = plsc.VectorSubcoreMesh(core_axis_name="c", subcore_axis_name="s")
    kern = functools.partial(
        pl.kernel,
        mesh=mesh,
        out_type=jax.ShapeDtypeStruct((NC, N_PAD, D), jnp.float32),
        scratch_types=(
            [pltpu.VMEM((CH,), jnp.int32)] * (2 * RING)
            + [pltpu.VMEM((CH, D), jnp.float32)] * RING
            + [pltpu.VMEM_SHARED((N_PAD, D), jnp.float32)]
            + [pltpu.SemaphoreType.DMA] * (3 * RING)
        ),
    )(_seg_body)
    return kern(x_pad, row, col)


def _fin_body(p_ref, x_ref, w_ref, b_ref, o_ref):
    acc = p_ref[0] + p_ref[1] - x_ref[...]
    o_ref[...] = jnp.dot(acc, w_ref[...],
                         preferred_element_type=jnp.float32) + b_ref[...]


def _finish(partials, x_pad, W, b):
    blk = 1024
    return pl.pallas_call(
        _fin_body,
        grid=(N_PAD // blk,),
        in_specs=[
            pl.BlockSpec((NC, blk, D), lambda i: (0, i, 0)),
            pl.BlockSpec((blk, D), lambda i: (i, 0)),
            pl.BlockSpec((D, D), lambda i: (0, 0)),
            pl.BlockSpec((1, D), lambda i: (0, 0)),
        ],
        out_specs=pl.BlockSpec((blk, D), lambda i: (i, 0)),
        out_shape=jax.ShapeDtypeStruct((N_PAD, D), jnp.float32),
    )(partials, x_pad, W, b)


def kernel(x, adj, W, b):
    row = adj[0].astype(jnp.int32)
    col = adj[1].astype(jnp.int32)
    x_pad = jnp.pad(x, ((0, N_PAD - N), (0, 0)))
    partials = _segment_partials(x_pad, row, col)
    return _finish(partials, x_pad, W, b[None, :])[:N]


# speedup vs baseline: 14.3929x; 1.0571x over previous
import functools

import jax
import jax.numpy as jnp
from jax import lax
from jax.experimental import pallas as pl
from jax.experimental.pallas import tpu as pltpu
from jax.experimental.pallas import tpu_sc as plsc

N = 10000
E = 320000
D = 128

NC = 2
NS = 16
CH = 80
N_PAD = 10240

E_PER_CORE = E // NC
E_PER_TILE = E_PER_CORE // NS
N_CHUNKS = E_PER_TILE // CH
ROWS_PER_TILE = N_PAD // NS
LAST_R0 = (NS - 1) * ROWS_PER_TILE
LAST_ROWS = N - LAST_R0


RING = 4


def _seg_body(x_hbm, row_hbm, col_hbm, out_hbm,
              r0v, r1v, r2v, r3v, c0v, c1v, c2v, c3v,
              b0, b1, b2, b3, agg_sh,
              sg0, sg1, sg2, sg3, sr0, sr1, sr2, sr3,
              sc0, sc1, sc2, sc3):
    cid = lax.axis_index("c")
    sid = lax.axis_index("s")

    r0 = sid * ROWS_PER_TILE
    base = cid * E_PER_CORE + sid * E_PER_TILE
    rows = (r0v, r1v, r2v, r3v)
    cols = (c0v, c1v, c2v, c3v)
    bufs = (b0, b1, b2, b3)
    gsem = (sg0, sg1, sg2, sg3)
    rsem = (sr0, sr1, sr2, sr3)
    csem = (sc0, sc1, sc2, sc3)

    gd, cd = [], []
    for k in range(RING):
        eb = base + k * CH
        pltpu.sync_copy(row_hbm.at[pl.ds(eb, CH)], rows[k])
        gd.append(pltpu.async_copy(x_hbm.at[rows[k]], bufs[k], gsem[k]))
        cd.append(pltpu.async_copy(col_hbm.at[pl.ds(eb, CH)], cols[k],
                                   csem[k]))

    @pl.when(sid < NS - 1)
    def _():
        pltpu.sync_copy(x_hbm.at[pl.ds(r0, ROWS_PER_TILE)],
                        agg_sh.at[pl.ds(r0, ROWS_PER_TILE)])

    @pl.when(sid == NS - 1)
    def _():
        pltpu.sync_copy(x_hbm.at[pl.ds(LAST_R0, LAST_ROWS)],
                        agg_sh.at[pl.ds(LAST_R0, LAST_ROWS)])

    plsc.subcore_barrier()

    def body(j, _):
        for t in range(RING):
            i = RING * j + t
            nxt = i + RING
            eb = base + nxt * CH
            live = nxt < N_CHUNKS

            gd[t].wait()
            cd[t].wait()

            @pl.when(live)
            def _():
                pltpu.async_copy(row_hbm.at[pl.ds(eb, CH)], rows[t],
                                 rsem[t])

            pltpu.sync_copy(bufs[t], agg_sh.at[cols[t]], add=True)

            @pl.when(live)
            def _():
                pltpu.make_async_copy(row_hbm.at[pl.ds(eb, CH)], rows[t],
                                      rsem[t]).wait()
                pltpu.async_copy(x_hbm.at[rows[t]], bufs[t], gsem[t])
                pltpu.async_copy(col_hbm.at[pl.ds(eb, CH)], cols[t],
                                 csem[t])
        return _

    lax.fori_loop(0, N_CHUNKS // RING, body, None)

    for t in range(N_CHUNKS % RING):
        gd[t].wait()
        cd[t].wait()
        pltpu.sync_copy(bufs[t], agg_sh.at[cols[t]], add=True)

    plsc.subcore_barrier()

    pltpu.sync_copy(agg_sh.at[pl.ds(r0, ROWS_PER_TILE)],
                    out_hbm.at[cid, pl.ds(r0, ROWS_PER_TILE)])


def _segment_partials(x_pad, row, col):
    mesh = plsc.VectorSubcoreMesh(core_axis_name="c", subcore_axis_name="s")
    kern = functools.partial(
        pl.kernel,
        mesh=mesh,
        out_type=jax.ShapeDtypeStruct((NC, N_PAD, D), jnp.float32),
        scratch_types=(
            [pltpu.VMEM((CH,), jnp.int32)] * (2 * RING)
            + [pltpu.VMEM((CH, D), jnp.float32)] * RING
            + [pltpu.VMEM_SHARED((N_PAD, D), jnp.float32)]
            + [pltpu.SemaphoreType.DMA] * (3 * RING)
        ),
    )(_seg_body)
    return kern(x_pad, row, col)


def _fin_body(p_ref, x_ref, w_ref, b_ref, o_ref):
    acc = p_ref[0] + p_ref[1] - x_ref[...]
    o_ref[...] = jnp.dot(acc, w_ref[...],
                         preferred_element_type=jnp.float32) + b_ref[...]


def _finish(partials, x, W, b):
    blk = 1000
    return pl.pallas_call(
        _fin_body,
        grid=(N // blk,),
        in_specs=[
            pl.BlockSpec((NC, blk, D), lambda i: (0, i, 0)),
            pl.BlockSpec((blk, D), lambda i: (i, 0)),
            pl.BlockSpec((D, D), lambda i: (0, 0)),
            pl.BlockSpec((1, D), lambda i: (0, 0)),
        ],
        out_specs=pl.BlockSpec((blk, D), lambda i: (i, 0)),
        out_shape=jax.ShapeDtypeStruct((N, D), jnp.float32),
    )(partials, x, W, b)


def kernel(x, adj, W, b):
    row = adj[0].astype(jnp.int32)
    col = adj[1].astype(jnp.int32)
    partials = _segment_partials(x, row, col)
    return _finish(partials, x, W, b[None, :])
